# baseline (speedup 1.0000x reference)
import math

import jax
import jax.numpy as jnp
from jax import lax
from jax.experimental import pallas as pl
from jax.experimental.pallas import tpu as pltpu
from jax.experimental.pallas import tpu_sc as plsc

N2 = 10240
CHUNK = 128
NC, NS = 2, 16
NW = NC * NS
ROWS = N2 // NS
IB = 16

_MESH = plsc.VectorSubcoreMesh(core_axis_name="c", subcore_axis_name="s")


def _hist_body(col_hbm, zrows_hbm, ones_hbm, out_hbm, colv, onesv, acc, sem):
    del sem
    c = lax.axis_index("c")
    s = lax.axis_index("s")
    wid = c * NS + s
    nch = col_hbm.shape[1]
    pltpu.sync_copy(zrows_hbm, acc.at[pl.ds(s * ROWS, ROWS)])
    pltpu.sync_copy(ones_hbm, onesv)
    pltpu.sync_copy(col_hbm.at[wid], colv)
    plsc.subcore_barrier()

    def step(j, carry):
        pltpu.sync_copy(onesv, acc.at[colv.at[j]], add=True)
        return carry

    lax.fori_loop(0, nch, step, 0)
    plsc.subcore_barrier()
    pltpu.sync_copy(acc.at[pl.ds(s * ROWS, ROWS)],
                    out_hbm.at[c, pl.ds(s * ROWS, ROWS)])


def _scatter_body(row_hbm, col_hbm, y_hbm, ztile_hbm, out_hbm,
                  rowv, colv, bufs, acc, gsem0, gsem1, isem):
    c = lax.axis_index("c")
    s = lax.axis_index("s")
    wid = c * NS + s
    nch = row_hbm.shape[1]
    nblk = nch // IB
    pltpu.sync_copy(ztile_hbm, acc.at[pl.ds(s * ROWS, ROWS)])
    row_w = row_hbm.at[wid]
    col_w = col_hbm.at[wid]
    pltpu.sync_copy(row_w.at[pl.ds(0, IB)], rowv.at[0])
    pltpu.sync_copy(col_w.at[pl.ds(0, IB)], colv.at[0])
    pltpu.async_copy(row_w.at[pl.ds(IB, IB)], rowv.at[1], isem)
    pltpu.async_copy(col_w.at[pl.ds(IB, IB)], colv.at[1], isem)
    plsc.subcore_barrier()

    y_c = y_hbm.at[c]
    pltpu.async_copy(y_c.at[rowv.at[0, 0]], bufs.at[0], gsem0)
    pltpu.async_copy(y_c.at[rowv.at[0, 1]], bufs.at[1], gsem1)

    def step(j, carry):
        bi = j // IB
        jj = j % IB
        slot = bi % 2

        @pl.when((jj == 0) & (bi >= 1) & (bi + 1 < nblk))
        def _():
            nslot = (bi + 1) % 2
            pltpu.async_copy(row_w.at[pl.ds((bi + 1) * IB, IB)],
                             rowv.at[nslot], isem)
            pltpu.async_copy(col_w.at[pl.ds((bi + 1) * IB, IB)],
                             colv.at[nslot], isem)

        @pl.when((jj == IB - 2) & (bi + 1 < nblk))
        def _():
            nslot = (bi + 1) % 2
            pltpu.make_async_copy(row_w.at[pl.ds((bi + 1) * IB, IB)],
                                  rowv.at[nslot], isem).wait()
            pltpu.make_async_copy(col_w.at[pl.ds((bi + 1) * IB, IB)],
                                  colv.at[nslot], isem).wait()

        def chunk(p, gsem):
            pltpu.make_async_copy(y_c.at[rowv.at[slot, jj]],
                                  bufs.at[p], gsem).wait()
            pltpu.sync_copy(bufs.at[p], acc.at[colv.at[slot, jj]], add=True)

            @pl.when(j + 2 < nch)
            def _():
                j2 = j + 2
                pltpu.async_copy(
                    y_c.at[rowv.at[(j2 // IB) % 2, j2 % IB]],
                    bufs.at[p], gsem)

        @pl.when(j % 2 == 0)
        def _():
            chunk(0, gsem0)

        @pl.when(j % 2 == 1)
        def _():
            chunk(1, gsem1)

        return carry

    lax.fori_loop(0, nch, step, 0)
    plsc.subcore_barrier()
    pltpu.sync_copy(acc.at[pl.ds(s * ROWS, ROWS)],
                    out_hbm.at[c, pl.ds(s * ROWS, ROWS)])


def _scale_body(x_ref, w_ref, h0_ref, h1_ref, y_ref):
    deg = h0_ref[...] + h1_ref[...] + 1.0
    y_ref[...] = jnp.dot(x_ref[...], w_ref[...],
                         preferred_element_type=jnp.float32) * lax.rsqrt(deg)


def _final_body(p0_ref, p1_ref, y_ref, h0_ref, h1_ref, b_ref, o_ref):
    deg = h0_ref[...] + h1_ref[...] + 1.0
    t = (p0_ref[...] + p1_ref[...] + y_ref[...]) * lax.rsqrt(deg) + b_ref[...]
    o_ref[...] = t * 0.5 * (1.0 + lax.erf(t * (1.0 / math.sqrt(2.0))))


def kernel(x, edge_index, W, b):
    n, d = x.shape
    e = edge_index.shape[1]
    row = edge_index[0].astype(jnp.int32)
    col = edge_index[1].astype(jnp.int32)

    step = NW * CHUNK
    nch = ((e + step - 1) // step + IB - 1) // IB * IB
    e_pad = step * nch
    row3 = jnp.full((e_pad,), n, jnp.int32).at[:e].set(row).reshape(NW, nch, CHUNK)
    col3 = jnp.full((e_pad,), n, jnp.int32).at[:e].set(col).reshape(NW, nch, CHUNK)
    xpad = jnp.zeros((N2, d), jnp.float32).at[:n, :].set(x.astype(jnp.float32))

    zrows = jnp.zeros((ROWS,), jnp.float32)
    ones = jnp.ones((CHUNK,), jnp.float32)
    ztile = jnp.zeros((ROWS, d), jnp.float32)

    hist = pl.kernel(
        _hist_body,
        out_type=jax.ShapeDtypeStruct((NC, N2), jnp.float32),
        mesh=_MESH,
        scratch_types=[
            pltpu.VMEM((nch, CHUNK), jnp.int32),
            pltpu.VMEM((CHUNK,), jnp.float32),
            pltpu.VMEM_SHARED((N2,), jnp.float32),
            pltpu.SemaphoreType.DMA,
        ],
    )(col3, zrows, ones)

    h0 = hist[0].reshape(N2, 1)
    h1 = hist[1].reshape(N2, 1)

    blk = 1024
    y = pl.pallas_call(
        _scale_body,
        grid=(N2 // blk,),
        in_specs=[
            pl.BlockSpec((blk, d), lambda i: (i, 0)),
            pl.BlockSpec((d, d), lambda i: (0, 0)),
            pl.BlockSpec((blk, 1), lambda i: (i, 0)),
            pl.BlockSpec((blk, 1), lambda i: (i, 0)),
        ],
        out_specs=pl.BlockSpec((blk, d), lambda i: (i, 0)),
        out_shape=jax.ShapeDtypeStruct((N2, d), jnp.float32),
    )(xpad, W.astype(jnp.float32), h0, h1)

    y2 = jnp.stack([y, y])
    parts = pl.kernel(
        _scatter_body,
        out_type=jax.ShapeDtypeStruct((NC, N2, d), jnp.float32),
        mesh=_MESH,
        scratch_types=[
            pltpu.VMEM((2, IB, CHUNK), jnp.int32),
            pltpu.VMEM((2, IB, CHUNK), jnp.int32),
            pltpu.VMEM((2, CHUNK, d), jnp.float32),
            pltpu.VMEM_SHARED((N2, d), jnp.float32),
            pltpu.SemaphoreType.DMA,
            pltpu.SemaphoreType.DMA,
            pltpu.SemaphoreType.DMA,
        ],
    )(row3, col3, y2, ztile)

    out = pl.pallas_call(
        _final_body,
        grid=(N2 // blk,),
        in_specs=[
            pl.BlockSpec((blk, d), lambda i: (i, 0)),
            pl.BlockSpec((blk, d), lambda i: (i, 0)),
            pl.BlockSpec((blk, d), lambda i: (i, 0)),
            pl.BlockSpec((blk, 1), lambda i: (i, 0)),
            pl.BlockSpec((blk, 1), lambda i: (i, 0)),
            pl.BlockSpec((1, d), lambda i: (0, 0)),
        ],
        out_specs=pl.BlockSpec((blk, d), lambda i: (i, 0)),
        out_shape=jax.ShapeDtypeStruct((N2, d), jnp.float32),
    )(parts[0], parts[1], y, h0, h1, b.reshape(1, d).astype(jnp.float32))

    return out[:n]


# speedup vs baseline: 3.3547x; 3.3547x over previous
import math

import jax
import jax.numpy as jnp
from jax import lax
from jax.experimental import pallas as pl
from jax.experimental.pallas import tpu as pltpu
from jax.experimental.pallas import tpu_sc as plsc

N2 = 10240
CHUNK = 128
NC, NS = 2, 16
NW = NC * NS
ROWS = N2 // NS
IB = 16

_MESH = plsc.VectorSubcoreMesh(core_axis_name="c", subcore_axis_name="s")


def _hist_body(col_hbm, zrows_hbm, ones_hbm, out_hbm, colv, onesv, acc, sem):
    del sem
    c = lax.axis_index("c")
    s = lax.axis_index("s")
    wid = c * NS + s
    nch = col_hbm.shape[1]
    pltpu.sync_copy(zrows_hbm, acc.at[pl.ds(s * ROWS, ROWS)])
    pltpu.sync_copy(ones_hbm, onesv)
    pltpu.sync_copy(col_hbm.at[wid], colv)
    plsc.subcore_barrier()

    def step(j, carry):
        pltpu.sync_copy(onesv, acc.at[colv.at[j]], add=True)
        return carry

    lax.fori_loop(0, nch, step, 0)
    plsc.subcore_barrier()
    pltpu.sync_copy(acc.at[pl.ds(s * ROWS, ROWS)],
                    out_hbm.at[c, pl.ds(s * ROWS, ROWS)])


def _scatter_body(row_hbm, col_hbm, y_hbm, ztile_hbm, out_hbm,
                  rowv, colv, bufs, acc, gsem0, gsem1, isem):
    c = lax.axis_index("c")
    s = lax.axis_index("s")
    wid = c * NS + s
    nch = row_hbm.shape[1]
    nblk = nch // IB
    pltpu.sync_copy(ztile_hbm, acc.at[pl.ds(s * ROWS, ROWS)])
    row_w = row_hbm.at[wid]
    col_w = col_hbm.at[wid]
    pltpu.sync_copy(row_w.at[pl.ds(0, IB)], rowv.at[0])
    pltpu.sync_copy(col_w.at[pl.ds(0, IB)], colv.at[0])
    pltpu.async_copy(row_w.at[pl.ds(IB, IB)], rowv.at[1], isem)
    pltpu.async_copy(col_w.at[pl.ds(IB, IB)], colv.at[1], isem)
    plsc.subcore_barrier()

    y_c = y_hbm
    pltpu.async_copy(y_c.at[rowv.at[0, 0]], bufs.at[0], gsem0)
    pltpu.async_copy(y_c.at[rowv.at[0, 1]], bufs.at[1], gsem1)

    def step(j, carry):
        bi = j // IB
        jj = j % IB
        slot = bi % 2

        @pl.when((jj == 0) & (bi >= 1) & (bi + 1 < nblk))
        def _():
            nslot = (bi + 1) % 2
            pltpu.async_copy(row_w.at[pl.ds((bi + 1) * IB, IB)],
                             rowv.at[nslot], isem)
            pltpu.async_copy(col_w.at[pl.ds((bi + 1) * IB, IB)],
                             colv.at[nslot], isem)

        @pl.when((jj == IB - 2) & (bi + 1 < nblk))
        def _():
            nslot = (bi + 1) % 2
            pltpu.make_async_copy(row_w.at[pl.ds((bi + 1) * IB, IB)],
                                  rowv.at[nslot], isem).wait()
            pltpu.make_async_copy(col_w.at[pl.ds((bi + 1) * IB, IB)],
                                  colv.at[nslot], isem).wait()

        def chunk(p, gsem):
            pltpu.make_async_copy(y_c.at[rowv.at[slot, jj]],
                                  bufs.at[p], gsem).wait()
            pltpu.sync_copy(bufs.at[p], acc.at[colv.at[slot, jj]], add=True)

            @pl.when(j + 2 < nch)
            def _():
                j2 = j + 2
                pltpu.async_copy(
                    y_c.at[rowv.at[(j2 // IB) % 2, j2 % IB]],
                    bufs.at[p], gsem)

        @pl.when(j % 2 == 0)
        def _():
            chunk(0, gsem0)

        @pl.when(j % 2 == 1)
        def _():
            chunk(1, gsem1)

        return carry

    lax.fori_loop(0, nch, step, 0)
    plsc.subcore_barrier()
    pltpu.sync_copy(acc.at[pl.ds(s * ROWS, ROWS)],
                    out_hbm.at[c, pl.ds(s * ROWS, ROWS)])


def _scale_body(x_ref, w_ref, h0_ref, h1_ref, y_ref):
    deg = h0_ref[...] + h1_ref[...] + 1.0
    y_ref[...] = jnp.dot(x_ref[...], w_ref[...],
                         preferred_element_type=jnp.float32) * lax.rsqrt(deg)


def _final_body(p0_ref, p1_ref, y_ref, h0_ref, h1_ref, b_ref, o_ref):
    deg = h0_ref[...] + h1_ref[...] + 1.0
    t = (p0_ref[...] + p1_ref[...] + y_ref[...]) * lax.rsqrt(deg) + b_ref[...]
    o_ref[...] = t * 0.5 * (1.0 + lax.erf(t * (1.0 / math.sqrt(2.0))))


def kernel(x, edge_index, W, b):
    n, d = x.shape
    e = edge_index.shape[1]
    row = edge_index[0].astype(jnp.int32)
    col = edge_index[1].astype(jnp.int32)

    step = NW * CHUNK
    nch = ((e + step - 1) // step + IB - 1) // IB * IB
    e_pad = step * nch
    pad_iota = jnp.arange(e_pad, dtype=jnp.int32)
    prow = pad_iota % n
    pcol = n + pad_iota % (N2 - n)
    row3 = jnp.where(pad_iota < e, jnp.zeros((e_pad,), jnp.int32).at[:e].set(row),
                     prow).reshape(NW, nch, CHUNK)
    col3 = jnp.where(pad_iota < e, jnp.zeros((e_pad,), jnp.int32).at[:e].set(col),
                     pcol).reshape(NW, nch, CHUNK)
    xpad = jnp.zeros((N2, d), jnp.float32).at[:n, :].set(x.astype(jnp.float32))

    zrows = jnp.zeros((ROWS,), jnp.float32)
    ones = jnp.ones((CHUNK,), jnp.float32)
    ztile = jnp.zeros((ROWS, d), jnp.float32)

    hist = pl.kernel(
        _hist_body,
        out_type=jax.ShapeDtypeStruct((NC, N2), jnp.float32),
        mesh=_MESH,
        scratch_types=[
            pltpu.VMEM((nch, CHUNK), jnp.int32),
            pltpu.VMEM((CHUNK,), jnp.float32),
            pltpu.VMEM_SHARED((N2,), jnp.float32),
            pltpu.SemaphoreType.DMA,
        ],
    )(col3, zrows, ones)

    h0 = hist[0].reshape(N2, 1)
    h1 = hist[1].reshape(N2, 1)

    blk = 1024
    y = pl.pallas_call(
        _scale_body,
        grid=(N2 // blk,),
        in_specs=[
            pl.BlockSpec((blk, d), lambda i: (i, 0)),
            pl.BlockSpec((d, d), lambda i: (0, 0)),
            pl.BlockSpec((blk, 1), lambda i: (i, 0)),
            pl.BlockSpec((blk, 1), lambda i: (i, 0)),
        ],
        out_specs=pl.BlockSpec((blk, d), lambda i: (i, 0)),
        out_shape=jax.ShapeDtypeStruct((N2, d), jnp.float32),
    )(xpad, W.astype(jnp.float32), h0, h1)

    parts = pl.kernel(
        _scatter_body,
        out_type=jax.ShapeDtypeStruct((NC, N2, d), jnp.float32),
        mesh=_MESH,
        scratch_types=[
            pltpu.VMEM((2, IB, CHUNK), jnp.int32),
            pltpu.VMEM((2, IB, CHUNK), jnp.int32),
            pltpu.VMEM((2, CHUNK, d), jnp.float32),
            pltpu.VMEM_SHARED((N2, d), jnp.float32),
            pltpu.SemaphoreType.DMA,
            pltpu.SemaphoreType.DMA,
            pltpu.SemaphoreType.DMA,
        ],
    )(row3, col3, y, ztile)

    out = pl.pallas_call(
        _final_body,
        grid=(N2 // blk,),
        in_specs=[
            pl.BlockSpec((blk, d), lambda i: (i, 0)),
            pl.BlockSpec((blk, d), lambda i: (i, 0)),
            pl.BlockSpec((blk, d), lambda i: (i, 0)),
            pl.BlockSpec((blk, 1), lambda i: (i, 0)),
            pl.BlockSpec((blk, 1), lambda i: (i, 0)),
            pl.BlockSpec((1, d), lambda i: (0, 0)),
        ],
        out_specs=pl.BlockSpec((blk, d), lambda i: (i, 0)),
        out_shape=jax.ShapeDtypeStruct((N2, d), jnp.float32),
    )(parts[0], parts[1], y, h0, h1, b.reshape(1, d).astype(jnp.float32))

    return out[:n]


# speedup vs baseline: 3.5752x; 1.0657x over previous
import math

import jax
import jax.numpy as jnp
from jax import lax
from jax.experimental import pallas as pl
from jax.experimental.pallas import tpu as pltpu
from jax.experimental.pallas import tpu_sc as plsc

N2 = 10240
CHUNK = 128
NC, NS = 2, 16
NW = NC * NS
ROWS = N2 // NS
IB = 16

_MESH = plsc.VectorSubcoreMesh(core_axis_name="c", subcore_axis_name="s")


def _hist_body(col_hbm, zrows_hbm, ones_hbm, out_hbm, colv, onesv, acc, sem):
    c = lax.axis_index("c")
    s = lax.axis_index("s")
    wid = c * NS + s
    nch = col_hbm.shape[1]
    pltpu.sync_copy(zrows_hbm, acc.at[pl.ds(s * ROWS, ROWS)])
    pltpu.sync_copy(ones_hbm, onesv)
    pltpu.sync_copy(col_hbm.at[wid], colv)
    plsc.subcore_barrier()

    def fire(j, carry):
        pltpu.sync_copy(onesv, acc.at[colv.at[j]], add=True)
        return carry

    lax.fori_loop(0, nch, fire, 0)
    plsc.subcore_barrier()
    pltpu.sync_copy(acc.at[pl.ds(s * ROWS, ROWS)],
                    out_hbm.at[c, pl.ds(s * ROWS, ROWS)])


def _scatter_body(row_hbm, col_hbm, y_hbm, ztile_hbm, out_hbm,
                  rowv, colv, bufs, acc, gsem0, gsem1, isem):
    c = lax.axis_index("c")
    s = lax.axis_index("s")
    wid = c * NS + s
    nch = row_hbm.shape[1]
    nblk = nch // IB
    pltpu.sync_copy(ztile_hbm, acc.at[pl.ds(s * ROWS, ROWS)])
    row_w = row_hbm.at[wid]
    col_w = col_hbm.at[wid]
    pltpu.sync_copy(row_w.at[pl.ds(0, IB)], rowv.at[0])
    pltpu.sync_copy(col_w.at[pl.ds(0, IB)], colv.at[0])
    pltpu.async_copy(row_w.at[pl.ds(IB, IB)], rowv.at[1], isem)
    pltpu.async_copy(col_w.at[pl.ds(IB, IB)], colv.at[1], isem)
    plsc.subcore_barrier()

    pltpu.async_copy(y_hbm.at[rowv.at[0, 0]], bufs.at[0], gsem0)
    pltpu.async_copy(y_hbm.at[rowv.at[0, 1]], bufs.at[1], gsem1)

    def step(j, carry):
        bi = j // IB
        jj = j % IB
        slot = bi % 2

        @pl.when((jj == 0) & (bi >= 1) & (bi + 1 < nblk))
        def _():
            nslot = (bi + 1) % 2
            pltpu.async_copy(row_w.at[pl.ds((bi + 1) * IB, IB)],
                             rowv.at[nslot], isem)
            pltpu.async_copy(col_w.at[pl.ds((bi + 1) * IB, IB)],
                             colv.at[nslot], isem)

        @pl.when((jj == IB - 2) & (bi + 1 < nblk))
        def _():
            nslot = (bi + 1) % 2
            pltpu.make_async_copy(row_w.at[pl.ds((bi + 1) * IB, IB)],
                                  rowv.at[nslot], isem).wait()
            pltpu.make_async_copy(col_w.at[pl.ds((bi + 1) * IB, IB)],
                                  colv.at[nslot], isem).wait()

        def chunk(p, gsem):
            pltpu.make_async_copy(y_hbm.at[rowv.at[slot, jj]],
                                  bufs.at[p], gsem).wait()
            pltpu.sync_copy(bufs.at[p], acc.at[colv.at[slot, jj]], add=True)

            @pl.when(j + 2 < nch)
            def _():
                j2 = j + 2
                pltpu.async_copy(
                    y_hbm.at[rowv.at[(j2 // IB) % 2, j2 % IB]],
                    bufs.at[p], gsem)

        @pl.when(j % 2 == 0)
        def _():
            chunk(0, gsem0)

        @pl.when(j % 2 == 1)
        def _():
            chunk(1, gsem1)

        return carry

    lax.fori_loop(0, nch, step, 0)
    plsc.subcore_barrier()
    pltpu.sync_copy(acc.at[pl.ds(s * ROWS, ROWS)],
                    out_hbm.at[c, pl.ds(s * ROWS, ROWS)])


def _matmul_body(x_ref, w_ref, xw_ref):
    xw_ref[...] = jnp.dot(x_ref[...], w_ref[...],
                          preferred_element_type=jnp.float32)


def _scale_body(xw_ref, h0_ref, h1_ref, y_ref):
    deg = h0_ref[...] + h1_ref[...] + 1.0
    y_ref[...] = xw_ref[...] * lax.rsqrt(deg)


def _final_body(p0_ref, p1_ref, y_ref, h0_ref, h1_ref, b_ref, o_ref):
    deg = h0_ref[...] + h1_ref[...] + 1.0
    t = ((p0_ref[0] + p1_ref[0] + y_ref[...]) * lax.rsqrt(deg)
         + b_ref[...])
    o_ref[...] = t * 0.5 * (1.0 + lax.erf(t * (1.0 / math.sqrt(2.0))))


def kernel(x, edge_index, W, b):
    n, d = x.shape
    e = edge_index.shape[1]
    row = edge_index[0].astype(jnp.int32)
    col = edge_index[1].astype(jnp.int32)

    step = NW * CHUNK
    nch = ((e + step - 1) // step + IB - 1) // IB * IB
    e_pad = step * nch
    npad = e_pad - e
    pad_iota = jnp.arange(npad, dtype=jnp.int32)
    row3 = jnp.concatenate([row, pad_iota % n]).reshape(NW, nch, CHUNK)
    col3 = jnp.concatenate([col, n + pad_iota % (N2 - n)]).reshape(NW, nch, CHUNK)
    xpad = jnp.zeros((N2, d), jnp.float32).at[:n, :].set(x.astype(jnp.float32))

    zrows = jnp.zeros((ROWS,), jnp.float32)
    ones = jnp.ones((CHUNK,), jnp.float32)
    ztile = jnp.zeros((ROWS, d), jnp.float32)

    blk = 1024
    xw = pl.pallas_call(
        _matmul_body,
        grid=(N2 // blk,),
        in_specs=[
            pl.BlockSpec((blk, d), lambda i: (i, 0)),
            pl.BlockSpec((d, d), lambda i: (0, 0)),
        ],
        out_specs=pl.BlockSpec((blk, d), lambda i: (i, 0)),
        out_shape=jax.ShapeDtypeStruct((N2, d), jnp.float32),
    )(xpad, W.astype(jnp.float32))

    hist = pl.kernel(
        _hist_body,
        out_type=jax.ShapeDtypeStruct((NC, N2), jnp.float32),
        mesh=_MESH,
        scratch_types=[
            pltpu.VMEM((nch, CHUNK), jnp.int32),
            pltpu.VMEM((CHUNK,), jnp.float32),
            pltpu.VMEM_SHARED((N2,), jnp.float32),
            pltpu.SemaphoreType.DMA,
        ],
    )(col3, zrows, ones)

    h0 = hist[0].reshape(N2, 1)
    h1 = hist[1].reshape(N2, 1)

    y = pl.pallas_call(
        _scale_body,
        grid=(N2 // blk,),
        in_specs=[
            pl.BlockSpec((blk, d), lambda i: (i, 0)),
            pl.BlockSpec((blk, 1), lambda i: (i, 0)),
            pl.BlockSpec((blk, 1), lambda i: (i, 0)),
        ],
        out_specs=pl.BlockSpec((blk, d), lambda i: (i, 0)),
        out_shape=jax.ShapeDtypeStruct((N2, d), jnp.float32),
    )(xw, h0, h1)

    parts = pl.kernel(
        _scatter_body,
        out_type=jax.ShapeDtypeStruct((NC, N2, d), jnp.float32),
        mesh=_MESH,
        scratch_types=[
            pltpu.VMEM((2, IB, CHUNK), jnp.int32),
            pltpu.VMEM((2, IB, CHUNK), jnp.int32),
            pltpu.VMEM((2, CHUNK, d), jnp.float32),
            pltpu.VMEM_SHARED((N2, d), jnp.float32),
            pltpu.SemaphoreType.DMA,
            pltpu.SemaphoreType.DMA,
            pltpu.SemaphoreType.DMA,
        ],
    )(row3, col3, y, ztile)

    blkf = 1000
    out = pl.pallas_call(
        _final_body,
        grid=(n // blkf,),
        in_specs=[
            pl.BlockSpec((1, blkf, d), lambda i: (0, i, 0)),
            pl.BlockSpec((1, blkf, d), lambda i: (1, i, 0)),
            pl.BlockSpec((blkf, d), lambda i: (i, 0)),
            pl.BlockSpec((blkf, 1), lambda i: (i, 0)),
            pl.BlockSpec((blkf, 1), lambda i: (i, 0)),
            pl.BlockSpec((1, d), lambda i: (0, 0)),
        ],
        out_specs=pl.BlockSpec((blkf, d), lambda i: (i, 0)),
        out_shape=jax.ShapeDtypeStruct((n, d), jnp.float32),
    )(parts, parts, y, h0, h1, b.reshape(1, d).astype(jnp.float32))

    return out


# speedup vs baseline: 3.6265x; 1.0144x over previous
import math

import jax
import jax.numpy as jnp
from jax import lax
from jax.experimental import pallas as pl
from jax.experimental.pallas import tpu as pltpu
from jax.experimental.pallas import tpu_sc as plsc

N2 = 10240
CHUNK = 128
NC, NS = 2, 16
NW = NC * NS
ROWS = N2 // NS
IB = 16

_MESH = plsc.VectorSubcoreMesh(core_axis_name="c", subcore_axis_name="s")


def _hist_body(col_hbm, zrows_hbm, ones_hbm, out_hbm, colv, onesv, acc, sem):
    c = lax.axis_index("c")
    s = lax.axis_index("s")
    wid = c * NS + s
    nch = col_hbm.shape[1]
    pltpu.sync_copy(zrows_hbm, acc.at[pl.ds(s * ROWS, ROWS)])
    pltpu.sync_copy(ones_hbm, onesv)
    pltpu.sync_copy(col_hbm.at[wid], colv)
    plsc.subcore_barrier()

    def fire(j, carry):
        pltpu.async_copy(onesv, acc.at[colv.at[j]], sem, add=True)
        return carry

    lax.fori_loop(0, nch, fire, 0)

    def drain(j, carry):
        pltpu.make_async_copy(onesv, acc.at[colv.at[j]], sem).wait()
        return carry

    lax.fori_loop(0, nch, drain, 0)
    plsc.subcore_barrier()
    pltpu.sync_copy(acc.at[pl.ds(s * ROWS, ROWS)],
                    out_hbm.at[c, pl.ds(s * ROWS, ROWS)])


def _scatter_body(row_hbm, col_hbm, y_hbm, ztile_hbm, out_hbm,
                  rowv, colv, bufs, acc, gsem0, gsem1, isem):
    c = lax.axis_index("c")
    s = lax.axis_index("s")
    wid = c * NS + s
    nch = row_hbm.shape[1]
    nblk = nch // IB
    pltpu.sync_copy(ztile_hbm, acc.at[pl.ds(s * ROWS, ROWS)])
    row_w = row_hbm.at[wid]
    col_w = col_hbm.at[wid]
    pltpu.sync_copy(row_w.at[pl.ds(0, IB)], rowv.at[0])
    pltpu.sync_copy(col_w.at[pl.ds(0, IB)], colv.at[0])
    pltpu.async_copy(row_w.at[pl.ds(IB, IB)], rowv.at[1], isem)
    pltpu.async_copy(col_w.at[pl.ds(IB, IB)], colv.at[1], isem)
    plsc.subcore_barrier()

    pltpu.async_copy(y_hbm.at[rowv.at[0, 0]], bufs.at[0], gsem0)
    pltpu.async_copy(y_hbm.at[rowv.at[0, 1]], bufs.at[1], gsem1)

    def step(j, carry):
        bi = j // IB
        jj = j % IB
        slot = bi % 2

        @pl.when((jj == 0) & (bi >= 1) & (bi + 1 < nblk))
        def _():
            nslot = (bi + 1) % 2
            pltpu.async_copy(row_w.at[pl.ds((bi + 1) * IB, IB)],
                             rowv.at[nslot], isem)
            pltpu.async_copy(col_w.at[pl.ds((bi + 1) * IB, IB)],
                             colv.at[nslot], isem)

        @pl.when((jj == IB - 2) & (bi + 1 < nblk))
        def _():
            nslot = (bi + 1) % 2
            pltpu.make_async_copy(row_w.at[pl.ds((bi + 1) * IB, IB)],
                                  rowv.at[nslot], isem).wait()
            pltpu.make_async_copy(col_w.at[pl.ds((bi + 1) * IB, IB)],
                                  colv.at[nslot], isem).wait()

        def chunk(p, gsem):
            pltpu.make_async_copy(y_hbm.at[rowv.at[slot, jj]],
                                  bufs.at[p], gsem).wait()
            pltpu.sync_copy(bufs.at[p], acc.at[colv.at[slot, jj]], add=True)

            @pl.when(j + 2 < nch)
            def _():
                j2 = j + 2
                pltpu.async_copy(
                    y_hbm.at[rowv.at[(j2 // IB) % 2, j2 % IB]],
                    bufs.at[p], gsem)

        @pl.when(j % 2 == 0)
        def _():
            chunk(0, gsem0)

        @pl.when(j % 2 == 1)
        def _():
            chunk(1, gsem1)

        return carry

    lax.fori_loop(0, nch, step, 0)
    plsc.subcore_barrier()
    pltpu.sync_copy(acc.at[pl.ds(s * ROWS, ROWS)],
                    out_hbm.at[c, pl.ds(s * ROWS, ROWS)])


def _matmul_body(x_ref, w_ref, xw_ref):
    xw_ref[...] = jnp.dot(x_ref[...], w_ref[...],
                          preferred_element_type=jnp.float32)


def _scale_body(xw_ref, h0_ref, h1_ref, y_ref):
    deg = h0_ref[...] + h1_ref[...] + 1.0
    y_ref[...] = xw_ref[...] * lax.rsqrt(deg)


def _final_body(p0_ref, p1_ref, y_ref, h0_ref, h1_ref, b_ref, o_ref):
    deg = h0_ref[...] + h1_ref[...] + 1.0
    t = ((p0_ref[0] + p1_ref[0] + y_ref[...]) * lax.rsqrt(deg)
         + b_ref[...])
    o_ref[...] = t * 0.5 * (1.0 + lax.erf(t * (1.0 / math.sqrt(2.0))))


def kernel(x, edge_index, W, b):
    n, d = x.shape
    e = edge_index.shape[1]
    row = edge_index[0].astype(jnp.int32)
    col = edge_index[1].astype(jnp.int32)

    step = NW * CHUNK
    nch = ((e + step - 1) // step + IB - 1) // IB * IB
    e_pad = step * nch
    npad = e_pad - e
    pad_iota = jnp.arange(npad, dtype=jnp.int32)
    row3 = jnp.concatenate([row, pad_iota % n]).reshape(NW, nch, CHUNK)
    col3 = jnp.concatenate([col, n + pad_iota % (N2 - n)]).reshape(NW, nch, CHUNK)
    xpad = jnp.zeros((N2, d), jnp.float32).at[:n, :].set(x.astype(jnp.float32))

    zrows = jnp.zeros((ROWS,), jnp.float32)
    ones = jnp.ones((CHUNK,), jnp.float32)
    ztile = jnp.zeros((ROWS, d), jnp.float32)

    blk = 1024
    xw = pl.pallas_call(
        _matmul_body,
        grid=(N2 // blk,),
        in_specs=[
            pl.BlockSpec((blk, d), lambda i: (i, 0)),
            pl.BlockSpec((d, d), lambda i: (0, 0)),
        ],
        out_specs=pl.BlockSpec((blk, d), lambda i: (i, 0)),
        out_shape=jax.ShapeDtypeStruct((N2, d), jnp.float32),
    )(xpad, W.astype(jnp.float32))

    hist = pl.kernel(
        _hist_body,
        out_type=jax.ShapeDtypeStruct((NC, N2), jnp.float32),
        mesh=_MESH,
        scratch_types=[
            pltpu.VMEM((nch, CHUNK), jnp.int32),
            pltpu.VMEM((CHUNK,), jnp.float32),
            pltpu.VMEM_SHARED((N2,), jnp.float32),
            pltpu.SemaphoreType.DMA,
        ],
    )(col3, zrows, ones)

    h0 = hist[0].reshape(N2, 1)
    h1 = hist[1].reshape(N2, 1)

    y = pl.pallas_call(
        _scale_body,
        grid=(N2 // blk,),
        in_specs=[
            pl.BlockSpec((blk, d), lambda i: (i, 0)),
            pl.BlockSpec((blk, 1), lambda i: (i, 0)),
            pl.BlockSpec((blk, 1), lambda i: (i, 0)),
        ],
        out_specs=pl.BlockSpec((blk, d), lambda i: (i, 0)),
        out_shape=jax.ShapeDtypeStruct((N2, d), jnp.float32),
    )(xw, h0, h1)

    parts = pl.kernel(
        _scatter_body,
        out_type=jax.ShapeDtypeStruct((NC, N2, d), jnp.float32),
        mesh=_MESH,
        scratch_types=[
            pltpu.VMEM((2, IB, CHUNK), jnp.int32),
            pltpu.VMEM((2, IB, CHUNK), jnp.int32),
            pltpu.VMEM((2, CHUNK, d), jnp.float32),
            pltpu.VMEM_SHARED((N2, d), jnp.float32),
            pltpu.SemaphoreType.DMA,
            pltpu.SemaphoreType.DMA,
            pltpu.SemaphoreType.DMA,
        ],
    )(row3, col3, y, ztile)

    blkf = 1000
    out = pl.pallas_call(
        _final_body,
        grid=(n // blkf,),
        in_specs=[
            pl.BlockSpec((1, blkf, d), lambda i: (0, i, 0)),
            pl.BlockSpec((1, blkf, d), lambda i: (1, i, 0)),
            pl.BlockSpec((blkf, d), lambda i: (i, 0)),
            pl.BlockSpec((blkf, 1), lambda i: (i, 0)),
            pl.BlockSpec((blkf, 1), lambda i: (i, 0)),
            pl.BlockSpec((1, d), lambda i: (0, 0)),
        ],
        out_specs=pl.BlockSpec((blkf, d), lambda i: (i, 0)),
        out_shape=jax.ShapeDtypeStruct((n, d), jnp.float32),
    )(parts, parts, y, h0, h1, b.reshape(1, d).astype(jnp.float32))

    return out


# speedup vs baseline: 3.7647x; 1.0381x over previous
import math

import jax
import jax.numpy as jnp
from jax import lax
from jax.experimental import pallas as pl
from jax.experimental.pallas import tpu as pltpu
from jax.experimental.pallas import tpu_sc as plsc

N2 = 10240
HCHUNK = 128
CHUNK = 64
NSLOT = 4
NC, NS = 2, 16
NW = NC * NS
ROWS = N2 // NS
IB = 16

_MESH = plsc.VectorSubcoreMesh(core_axis_name="c", subcore_axis_name="s")


def _hist_body(col_hbm, zrows_hbm, ones_hbm, out_hbm, colv, onesv, acc, sem):
    c = lax.axis_index("c")
    s = lax.axis_index("s")
    wid = c * NS + s
    nch = col_hbm.shape[1]
    pltpu.sync_copy(zrows_hbm, acc.at[pl.ds(s * ROWS, ROWS)])
    pltpu.sync_copy(ones_hbm, onesv)
    pltpu.sync_copy(col_hbm.at[wid], colv)
    plsc.subcore_barrier()

    def fire(j, carry):
        pltpu.async_copy(onesv, acc.at[colv.at[j]], sem, add=True)
        return carry

    lax.fori_loop(0, nch, fire, 0)

    def drain(j, carry):
        pltpu.make_async_copy(onesv, acc.at[colv.at[j]], sem).wait()
        return carry

    lax.fori_loop(0, nch, drain, 0)
    plsc.subcore_barrier()
    pltpu.sync_copy(acc.at[pl.ds(s * ROWS, ROWS)],
                    out_hbm.at[c, pl.ds(s * ROWS, ROWS)])


def _scatter_body(row_hbm, col_hbm, y_hbm, ztile_hbm, out_hbm,
                  rowv, colv, bufs, acc, gs0, gs1, gs2, gs3,
                  ss0, ss1, ss2, ss3, isem):
    gsems = (gs0, gs1, gs2, gs3)
    ssems = (ss0, ss1, ss2, ss3)
    c = lax.axis_index("c")
    s = lax.axis_index("s")
    wid = c * NS + s
    nch = row_hbm.shape[1]
    nblk = nch // IB
    pltpu.sync_copy(ztile_hbm, acc.at[pl.ds(s * ROWS, ROWS)])
    row_w = row_hbm.at[wid]
    col_w = col_hbm.at[wid]
    pltpu.sync_copy(row_w.at[pl.ds(0, IB)], rowv.at[0])
    pltpu.sync_copy(col_w.at[pl.ds(0, IB)], colv.at[0])
    pltpu.async_copy(row_w.at[pl.ds(IB, IB)], rowv.at[1], isem)
    pltpu.async_copy(col_w.at[pl.ds(IB, IB)], colv.at[1], isem)
    plsc.subcore_barrier()

    for k in range(3):
        pltpu.async_copy(y_hbm.at[rowv.at[0, k]], bufs.at[k], gsems[k])

    def step(j, carry):
        bi = j // IB
        jj = j % IB
        slot = bi % 2

        @pl.when((jj == 1) & (bi >= 1) & (bi + 1 < nblk))
        def _():
            nslot = (bi + 1) % 2
            pltpu.async_copy(row_w.at[pl.ds((bi + 1) * IB, IB)],
                             rowv.at[nslot], isem)
            pltpu.async_copy(col_w.at[pl.ds((bi + 1) * IB, IB)],
                             colv.at[nslot], isem)

        @pl.when((jj == IB - 3) & (bi + 1 < nblk))
        def _():
            nslot = (bi + 1) % 2
            pltpu.make_async_copy(row_w.at[pl.ds((bi + 1) * IB, IB)],
                                  rowv.at[nslot], isem).wait()
            pltpu.make_async_copy(col_w.at[pl.ds((bi + 1) * IB, IB)],
                                  colv.at[nslot], isem).wait()

        def path(p):
            q = (p + 3) % NSLOT

            @pl.when(j + 3 < nch)
            def _():
                @pl.when(j >= 1)
                def _():
                    jm = j - 1
                    pltpu.make_async_copy(
                        bufs.at[q],
                        acc.at[colv.at[(jm // IB) % 2, jm % IB]],
                        ssems[q]).wait()
                j3 = j + 3
                pltpu.async_copy(
                    y_hbm.at[rowv.at[(j3 // IB) % 2, j3 % IB]],
                    bufs.at[q], gsems[q])

            pltpu.make_async_copy(y_hbm.at[rowv.at[slot, jj]],
                                  bufs.at[p], gsems[p]).wait()
            pltpu.async_copy(bufs.at[p], acc.at[colv.at[slot, jj]],
                             ssems[p], add=True)

        for p in range(NSLOT):
            @pl.when(j % NSLOT == p)
            def _(p=p):
                path(p)

        return carry

    lax.fori_loop(0, nch, step, 0)
    for k in range(-4, 0):
        kk = k
        pltpu.make_async_copy(bufs.at[(nch + kk) % NSLOT],
                              acc.at[colv.at[((nch + kk) // IB) % 2,
                                             (nch + kk) % IB]],
                              ssems[(nch + kk) % NSLOT]).wait()
    plsc.subcore_barrier()
    pltpu.sync_copy(acc.at[pl.ds(s * ROWS, ROWS)],
                    out_hbm.at[c, pl.ds(s * ROWS, ROWS)])


def _matmul_body(x_ref, w_ref, xw_ref):
    xw_ref[...] = jnp.dot(x_ref[...], w_ref[...],
                          preferred_element_type=jnp.float32)


def _scale_body(xw_ref, h0_ref, h1_ref, y_ref):
    deg = h0_ref[...] + h1_ref[...] + 1.0
    y_ref[...] = xw_ref[...] * lax.rsqrt(deg)


def _final_body(p0_ref, p1_ref, y_ref, h0_ref, h1_ref, b_ref, o_ref):
    deg = h0_ref[...] + h1_ref[...] + 1.0
    t = ((p0_ref[0] + p1_ref[0] + y_ref[...]) * lax.rsqrt(deg)
         + b_ref[...])
    o_ref[...] = t * 0.5 * (1.0 + lax.erf(t * (1.0 / math.sqrt(2.0))))


def kernel(x, edge_index, W, b):
    n, d = x.shape
    e = edge_index.shape[1]
    row = edge_index[0].astype(jnp.int32)
    col = edge_index[1].astype(jnp.int32)

    step = NW * CHUNK * IB
    e_pad = ((e + step - 1) // step) * step
    nch = e_pad // (NW * CHUNK)
    nch_h = e_pad // (NW * HCHUNK)
    npad = e_pad - e
    pad_iota = jnp.arange(npad, dtype=jnp.int32)
    row_p = jnp.concatenate([row, pad_iota % n])
    col_p = jnp.concatenate([col, n + pad_iota % (N2 - n)])
    row3 = row_p.reshape(NW, nch, CHUNK)
    col3 = col_p.reshape(NW, nch, CHUNK)
    col3h = col_p.reshape(NW, nch_h, HCHUNK)
    xpad = jnp.zeros((N2, d), jnp.float32).at[:n, :].set(x.astype(jnp.float32))

    zrows = jnp.zeros((ROWS,), jnp.float32)
    ones = jnp.ones((HCHUNK,), jnp.float32)
    ztile = jnp.zeros((ROWS, d), jnp.float32)

    blk = 1024
    xw = pl.pallas_call(
        _matmul_body,
        grid=(N2 // blk,),
        in_specs=[
            pl.BlockSpec((blk, d), lambda i: (i, 0)),
            pl.BlockSpec((d, d), lambda i: (0, 0)),
        ],
        out_specs=pl.BlockSpec((blk, d), lambda i: (i, 0)),
        out_shape=jax.ShapeDtypeStruct((N2, d), jnp.float32),
    )(xpad, W.astype(jnp.float32))

    hist = pl.kernel(
        _hist_body,
        out_type=jax.ShapeDtypeStruct((NC, N2), jnp.float32),
        mesh=_MESH,
        scratch_types=[
            pltpu.VMEM((nch_h, HCHUNK), jnp.int32),
            pltpu.VMEM((HCHUNK,), jnp.float32),
            pltpu.VMEM_SHARED((N2,), jnp.float32),
            pltpu.SemaphoreType.DMA,
        ],
    )(col3h, zrows, ones)

    h0 = hist[0].reshape(N2, 1)
    h1 = hist[1].reshape(N2, 1)

    y = pl.pallas_call(
        _scale_body,
        grid=(N2 // blk,),
        in_specs=[
            pl.BlockSpec((blk, d), lambda i: (i, 0)),
            pl.BlockSpec((blk, 1), lambda i: (i, 0)),
            pl.BlockSpec((blk, 1), lambda i: (i, 0)),
        ],
        out_specs=pl.BlockSpec((blk, d), lambda i: (i, 0)),
        out_shape=jax.ShapeDtypeStruct((N2, d), jnp.float32),
    )(xw, h0, h1)

    parts = pl.kernel(
        _scatter_body,
        out_type=jax.ShapeDtypeStruct((NC, N2, d), jnp.float32),
        mesh=_MESH,
        scratch_types=[
            pltpu.VMEM((2, IB, CHUNK), jnp.int32),
            pltpu.VMEM((2, IB, CHUNK), jnp.int32),
            pltpu.VMEM((NSLOT, CHUNK, d), jnp.float32),
            pltpu.VMEM_SHARED((N2, d), jnp.float32),
        ] + [pltpu.SemaphoreType.DMA] * 9,
    )(row3, col3, y, ztile)

    blkf = 1000
    out = pl.pallas_call(
        _final_body,
        grid=(n // blkf,),
        in_specs=[
            pl.BlockSpec((1, blkf, d), lambda i: (0, i, 0)),
            pl.BlockSpec((1, blkf, d), lambda i: (1, i, 0)),
            pl.BlockSpec((blkf, d), lambda i: (i, 0)),
            pl.BlockSpec((blkf, 1), lambda i: (i, 0)),
            pl.BlockSpec((blkf, 1), lambda i: (i, 0)),
            pl.BlockSpec((1, d), lambda i: (0, 0)),
        ],
        out_specs=pl.BlockSpec((blkf, d), lambda i: (i, 0)),
        out_shape=jax.ShapeDtypeStruct((n, d), jnp.float32),
    )(parts, parts, y, h0, h1, b.reshape(1, d).astype(jnp.float32))

    return out


# speedup vs baseline: 3.7967x; 1.0085x over previous
import math

import jax
import jax.numpy as jnp
from jax import lax
from jax.experimental import pallas as pl
from jax.experimental.pallas import tpu as pltpu
from jax.experimental.pallas import tpu_sc as plsc

N2 = 10240
HCHUNK = 128
CHUNK = 64
NSLOT = 5
AHEAD = NSLOT - 1
NC, NS = 2, 16
NW = NC * NS
ROWS = N2 // NS
IB = 16

_MESH = plsc.VectorSubcoreMesh(core_axis_name="c", subcore_axis_name="s")


def _hist_body(col_hbm, zrows_hbm, ones_hbm, out_hbm, colv, onesv, acc, sem):
    c = lax.axis_index("c")
    s = lax.axis_index("s")
    wid = c * NS + s
    nch = col_hbm.shape[1]
    pltpu.sync_copy(zrows_hbm, acc.at[pl.ds(s * ROWS, ROWS)])
    pltpu.sync_copy(ones_hbm, onesv)
    pltpu.sync_copy(col_hbm.at[wid], colv)
    plsc.subcore_barrier()

    def fire(j, carry):
        pltpu.async_copy(onesv, acc.at[colv.at[j]], sem, add=True)
        return carry

    lax.fori_loop(0, nch, fire, 0)

    pltpu.make_async_copy(col_hbm.at[wid], colv, sem).wait()
    plsc.subcore_barrier()
    pltpu.sync_copy(acc.at[pl.ds(s * ROWS, ROWS)],
                    out_hbm.at[c, pl.ds(s * ROWS, ROWS)])


def _scatter_body(row_hbm, col_hbm, y_hbm, ztile_hbm, out_hbm,
                  rowv, colv, bufs, acc, gs0, gs1, gs2, gs3, gs4,
                  ss0, ss1, ss2, ss3, ss4, isem):
    gsems = (gs0, gs1, gs2, gs3, gs4)
    ssems = (ss0, ss1, ss2, ss3, ss4)
    c = lax.axis_index("c")
    s = lax.axis_index("s")
    wid = c * NS + s
    nch = row_hbm.shape[1]
    nblk = nch // IB
    pltpu.sync_copy(ztile_hbm, acc.at[pl.ds(s * ROWS, ROWS)])
    row_w = row_hbm.at[wid]
    col_w = col_hbm.at[wid]
    pltpu.sync_copy(row_w.at[pl.ds(0, IB)], rowv.at[0])
    pltpu.sync_copy(col_w.at[pl.ds(0, IB)], colv.at[0])
    pltpu.async_copy(row_w.at[pl.ds(IB, IB)], rowv.at[1], isem)
    pltpu.async_copy(col_w.at[pl.ds(IB, IB)], colv.at[1], isem)
    plsc.subcore_barrier()

    for k in range(AHEAD):
        pltpu.async_copy(y_hbm.at[rowv.at[0, k]], bufs.at[k], gsems[k])

    def step(j, carry):
        bi = j // IB
        jj = j % IB
        slot = bi % 2

        @pl.when((jj == 1) & (bi >= 1) & (bi + 1 < nblk))
        def _():
            nslot = (bi + 1) % 2
            pltpu.async_copy(row_w.at[pl.ds((bi + 1) * IB, IB)],
                             rowv.at[nslot], isem)
            pltpu.async_copy(col_w.at[pl.ds((bi + 1) * IB, IB)],
                             colv.at[nslot], isem)

        @pl.when((jj == IB - AHEAD) & (bi + 1 < nblk))
        def _():
            nslot = (bi + 1) % 2
            pltpu.make_async_copy(row_w.at[pl.ds((bi + 1) * IB, IB)],
                                  rowv.at[nslot], isem).wait()
            pltpu.make_async_copy(col_w.at[pl.ds((bi + 1) * IB, IB)],
                                  colv.at[nslot], isem).wait()

        def path(p):
            q = (p + AHEAD) % NSLOT

            @pl.when(j + AHEAD < nch)
            def _():
                @pl.when(j >= 1)
                def _():
                    jm = j - 1
                    pltpu.make_async_copy(
                        bufs.at[q],
                        acc.at[colv.at[(jm // IB) % 2, jm % IB]],
                        ssems[q]).wait()
                j3 = j + AHEAD
                pltpu.async_copy(
                    y_hbm.at[rowv.at[(j3 // IB) % 2, j3 % IB]],
                    bufs.at[q], gsems[q])

            pltpu.make_async_copy(y_hbm.at[rowv.at[slot, jj]],
                                  bufs.at[p], gsems[p]).wait()
            pltpu.async_copy(bufs.at[p], acc.at[colv.at[slot, jj]],
                             ssems[p], add=True)

        for p in range(NSLOT):
            @pl.when(j % NSLOT == p)
            def _(p=p):
                path(p)

        return carry

    lax.fori_loop(0, nch, step, 0)
    for k in range(-(AHEAD + 1), 0):
        kk = k
        pltpu.make_async_copy(bufs.at[(nch + kk) % NSLOT],
                              acc.at[colv.at[((nch + kk) // IB) % 2,
                                             (nch + kk) % IB]],
                              ssems[(nch + kk) % NSLOT]).wait()
    plsc.subcore_barrier()
    pltpu.sync_copy(acc.at[pl.ds(s * ROWS, ROWS)],
                    out_hbm.at[c, pl.ds(s * ROWS, ROWS)])


def _matmul_body(x_ref, w_ref, xw_ref):
    xw_ref[...] = jnp.dot(x_ref[...], w_ref[...],
                          preferred_element_type=jnp.float32)


def _scale_body(xw_ref, h0_ref, h1_ref, y_ref):
    deg = h0_ref[...] + h1_ref[...] + 1.0
    y_ref[...] = xw_ref[...] * lax.rsqrt(deg)


def _final_body(p0_ref, p1_ref, y_ref, h0_ref, h1_ref, b_ref, o_ref):
    deg = h0_ref[...] + h1_ref[...] + 1.0
    t = ((p0_ref[0] + p1_ref[0] + y_ref[...]) * lax.rsqrt(deg)
         + b_ref[...])
    o_ref[...] = t * 0.5 * (1.0 + lax.erf(t * (1.0 / math.sqrt(2.0))))


def kernel(x, edge_index, W, b):
    n, d = x.shape
    e = edge_index.shape[1]
    row = edge_index[0].astype(jnp.int32)
    col = edge_index[1].astype(jnp.int32)

    step = NW * CHUNK * IB
    e_pad = ((e + step - 1) // step) * step
    nch = e_pad // (NW * CHUNK)
    nch_h = e_pad // (NW * HCHUNK)
    npad = e_pad - e
    pad_iota = jnp.arange(npad, dtype=jnp.int32)
    row_p = jnp.concatenate([row, pad_iota % n])
    col_p = jnp.concatenate([col, n + pad_iota % (N2 - n)])
    row3 = row_p.reshape(NW, nch, CHUNK)
    col3 = col_p.reshape(NW, nch, CHUNK)
    col3h = col_p.reshape(NW, nch_h, HCHUNK)
    xpad = jnp.zeros((N2, d), jnp.float32).at[:n, :].set(x.astype(jnp.float32))

    zrows = jnp.zeros((ROWS,), jnp.float32)
    ones = jnp.ones((HCHUNK,), jnp.float32)
    ztile = jnp.zeros((ROWS, d), jnp.float32)

    blk = 1024
    xw = pl.pallas_call(
        _matmul_body,
        grid=(N2 // blk,),
        in_specs=[
            pl.BlockSpec((blk, d), lambda i: (i, 0)),
            pl.BlockSpec((d, d), lambda i: (0, 0)),
        ],
        out_specs=pl.BlockSpec((blk, d), lambda i: (i, 0)),
        out_shape=jax.ShapeDtypeStruct((N2, d), jnp.float32),
    )(xpad, W.astype(jnp.float32))

    hist = pl.kernel(
        _hist_body,
        out_type=jax.ShapeDtypeStruct((NC, N2), jnp.float32),
        mesh=_MESH,
        scratch_types=[
            pltpu.VMEM((nch_h, HCHUNK), jnp.int32),
            pltpu.VMEM((HCHUNK,), jnp.float32),
            pltpu.VMEM_SHARED((N2,), jnp.float32),
            pltpu.SemaphoreType.DMA,
        ],
    )(col3h, zrows, ones)

    h0 = hist[0].reshape(N2, 1)
    h1 = hist[1].reshape(N2, 1)

    y = pl.pallas_call(
        _scale_body,
        grid=(N2 // blk,),
        in_specs=[
            pl.BlockSpec((blk, d), lambda i: (i, 0)),
            pl.BlockSpec((blk, 1), lambda i: (i, 0)),
            pl.BlockSpec((blk, 1), lambda i: (i, 0)),
        ],
        out_specs=pl.BlockSpec((blk, d), lambda i: (i, 0)),
        out_shape=jax.ShapeDtypeStruct((N2, d), jnp.float32),
    )(xw, h0, h1)

    parts = pl.kernel(
        _scatter_body,
        out_type=jax.ShapeDtypeStruct((NC, N2, d), jnp.float32),
        mesh=_MESH,
        scratch_types=[
            pltpu.VMEM((2, IB, CHUNK), jnp.int32),
            pltpu.VMEM((2, IB, CHUNK), jnp.int32),
            pltpu.VMEM((NSLOT, CHUNK, d), jnp.float32),
            pltpu.VMEM_SHARED((N2, d), jnp.float32),
        ] + [pltpu.SemaphoreType.DMA] * (2 * NSLOT + 1),
    )(row3, col3, y, ztile)

    blkf = 1000
    out = pl.pallas_call(
        _final_body,
        grid=(n // blkf,),
        in_specs=[
            pl.BlockSpec((1, blkf, d), lambda i: (0, i, 0)),
            pl.BlockSpec((1, blkf, d), lambda i: (1, i, 0)),
            pl.BlockSpec((blkf, d), lambda i: (i, 0)),
            pl.BlockSpec((blkf, 1), lambda i: (i, 0)),
            pl.BlockSpec((blkf, 1), lambda i: (i, 0)),
            pl.BlockSpec((1, d), lambda i: (0, 0)),
        ],
        out_specs=pl.BlockSpec((blkf, d), lambda i: (i, 0)),
        out_shape=jax.ShapeDtypeStruct((n, d), jnp.float32),
    )(parts, parts, y, h0, h1, b.reshape(1, d).astype(jnp.float32))

    return out
